# baseline (speedup 1.0000x reference)
import jax
import jax.numpy as jnp
from jax.experimental import pallas as pl
from jax.experimental.pallas import tpu as pltpu

N = 196
D = 768
B = 64
TH = 0.8
K2 = 3
MIN_N = 20
SCALE = 5.0
BIG = 2**30
_DN = (((1,), (1,)), ((), ()))


def _normalize(t):
    sq = t * t
    acc = jnp.sum(sq[:, :256], axis=1, keepdims=True)
    for k0 in range(256, D, 256):
        acc = acc + jnp.sum(sq[:, k0:k0 + 256], axis=1, keepdims=True)
    return t / jnp.maximum(jnp.sqrt(acc), 1e-12)


_LN2 = float(jnp.log(jnp.float32(2.0)))


def _entropy_row(mT):
    eT = jnp.exp(mT * SCALE)
    s = jnp.sum(eT, axis=0, keepdims=True)
    pT = eT / s
    return jnp.sum(-pT * (jnp.log2(pT) * _LN2), axis=0, keepdims=True)


def _adaptive_mask_col(ent_col, ones_col, ii, jj):
    emin = jnp.min(ent_col)
    emax = jnp.max(ent_col)
    abs_th = emin + (1.0 - TH) * (emax - emin)
    mask_col = ent_col < abs_th
    count = jnp.sum(mask_col.astype(jnp.float32))
    e_i = jax.lax.dot_general(ent_col, ones_col, _DN,
                              precision=jax.lax.Precision.HIGHEST,
                              preferred_element_type=jnp.float32)
    e_j = jax.lax.dot_general(ones_col, ent_col, _DN,
                              precision=jax.lax.Precision.HIGHEST,
                              preferred_element_type=jnp.float32)
    smaller = ((e_j < e_i) | ((e_j == e_i) & (jj < ii))).astype(jnp.float32)
    ranks_col = jax.lax.dot_general(smaller, ones_col, (((1,), (0,)), ((), ())),
                                    preferred_element_type=jnp.float32)
    fallback = (ranks_col < float(MIN_N)).astype(jnp.int32)
    return jnp.where(count < float(MIN_N), fallback, mask_col.astype(jnp.int32))


def _top3(aff, axis):
    idx_iota = jax.lax.broadcasted_iota(jnp.int32, (N, N), axis)
    work = aff
    cols = []
    for _ in range(K2):
        m = jnp.max(work, axis=axis, keepdims=True)
        idx = jnp.min(jnp.where(work == m, idx_iota, BIG), axis=axis)
        cols.append(idx[:, None])
        sel = idx[:, None] if axis == 1 else idx[None, :]
        work = jnp.where(idx_iota == sel, -jnp.inf, work)
    return jnp.concatenate(cols, axis=1)


def _image_kernel(q_ref, r_ref, aff_ref, entq_ref, entr_ref,
                  maskq_ref, maskr_ref, knnq_ref, knnr_ref):
    qn = _normalize(q_ref[0])
    rn = _normalize(r_ref[0])
    aff = jax.lax.dot_general(qn, rn, _DN,
                              preferred_element_type=jnp.float32)
    aff_ref[0] = aff

    ent_q = _entropy_row(aff.T)
    ent_r = _entropy_row(aff)
    entq_ref[0] = ent_q
    entr_ref[0] = ent_r

    ones_col = jnp.ones((N, 1), jnp.float32)
    ii = jax.lax.broadcasted_iota(jnp.int32, (N, N), 0)
    jj = jax.lax.broadcasted_iota(jnp.int32, (N, N), 1)
    maskq_ref[0] = _adaptive_mask_col(ent_q.T, ones_col, ii, jj).T
    maskr_ref[0] = _adaptive_mask_col(ent_r.T, ones_col, ii, jj).T

    knnq_ref[0] = _top3(aff, axis=1)
    knnr_ref[0] = _top3(aff, axis=0)


@jax.jit
def kernel(que_tokens, ref_tokens):
    grid = (B,)
    in_specs = [
        pl.BlockSpec((1, N, D), lambda b: (b, 0, 0)),
        pl.BlockSpec((1, N, D), lambda b: (b, 0, 0)),
    ]
    out_specs = [
        pl.BlockSpec((1, N, N), lambda b: (b, 0, 0)),
        pl.BlockSpec((1, 1, N), lambda b: (b, 0, 0)),
        pl.BlockSpec((1, 1, N), lambda b: (b, 0, 0)),
        pl.BlockSpec((1, 1, N), lambda b: (b, 0, 0)),
        pl.BlockSpec((1, 1, N), lambda b: (b, 0, 0)),
        pl.BlockSpec((1, N, K2), lambda b: (b, 0, 0)),
        pl.BlockSpec((1, N, K2), lambda b: (b, 0, 0)),
    ]
    out_shapes = [
        jax.ShapeDtypeStruct((B, N, N), jnp.float32),
        jax.ShapeDtypeStruct((B, 1, N), jnp.float32),
        jax.ShapeDtypeStruct((B, 1, N), jnp.float32),
        jax.ShapeDtypeStruct((B, 1, N), jnp.int32),
        jax.ShapeDtypeStruct((B, 1, N), jnp.int32),
        jax.ShapeDtypeStruct((B, N, K2), jnp.int32),
        jax.ShapeDtypeStruct((B, N, K2), jnp.int32),
    ]
    aff, ent_q, ent_r, mask_q, mask_r, knn_q2r, knn_r2q = pl.pallas_call(
        _image_kernel,
        grid=grid,
        in_specs=in_specs,
        out_specs=out_specs,
        out_shape=out_shapes,
        compiler_params=pltpu.CompilerParams(
            dimension_semantics=("parallel",),
        ),
    )(que_tokens, ref_tokens)
    return (aff,
            ent_q.reshape(B, N),
            ent_r.reshape(B, N),
            mask_q.reshape(B, N).astype(jnp.bool_),
            mask_r.reshape(B, N).astype(jnp.bool_),
            knn_q2r,
            knn_r2q)


# speedup vs baseline: 1.0006x; 1.0006x over previous
import math

import jax
import jax.numpy as jnp
from jax.experimental import pallas as pl
from jax.experimental.pallas import tpu as pltpu

N = 196
D = 768
B = 64
TH = 0.8
K2 = 3
MIN_N = 20
SCALE = 5.0
BIG = 2**30
_DN = (((1,), (1,)), ((), ()))


def _normalize(t):
    sq = t * t
    acc = jnp.sum(sq[:, :256], axis=1, keepdims=True)
    for k0 in range(256, D, 256):
        acc = acc + jnp.sum(sq[:, k0:k0 + 256], axis=1, keepdims=True)
    return t / jnp.maximum(jnp.sqrt(acc), 1e-12)


_LN2 = math.log(2.0)


def _entropy_row(mT):
    eT = jnp.exp(mT * SCALE)
    s = jnp.sum(eT, axis=0, keepdims=True)
    pT = eT / s
    return jnp.sum(-pT * (jnp.log2(pT) * _LN2), axis=0, keepdims=True)


def _adaptive_mask_col(ent_col, ones_col, ii, jj):
    emin = jnp.min(ent_col)
    emax = jnp.max(ent_col)
    abs_th = emin + (1.0 - TH) * (emax - emin)
    mask_col = ent_col < abs_th
    count = jnp.sum(mask_col.astype(jnp.float32))
    e_i = jax.lax.dot_general(ent_col, ones_col, _DN,
                              precision=jax.lax.Precision.HIGHEST,
                              preferred_element_type=jnp.float32)
    e_j = jax.lax.dot_general(ones_col, ent_col, _DN,
                              precision=jax.lax.Precision.HIGHEST,
                              preferred_element_type=jnp.float32)
    smaller = ((e_j < e_i) | ((e_j == e_i) & (jj < ii))).astype(jnp.float32)
    ranks_col = jax.lax.dot_general(smaller, ones_col, (((1,), (0,)), ((), ())),
                                    preferred_element_type=jnp.float32)
    fallback = (ranks_col < float(MIN_N)).astype(jnp.int32)
    return jnp.where(count < float(MIN_N), fallback, mask_col.astype(jnp.int32))


def _top3(aff, axis):
    idx_iota = jax.lax.broadcasted_iota(jnp.int32, (N, N), axis)
    work = aff
    cols = []
    for _ in range(K2):
        m = jnp.max(work, axis=axis, keepdims=True)
        idx = jnp.min(jnp.where(work == m, idx_iota, BIG), axis=axis)
        cols.append(idx[:, None])
        sel = idx[:, None] if axis == 1 else idx[None, :]
        work = jnp.where(idx_iota == sel, -jnp.inf, work)
    return jnp.concatenate(cols, axis=1)


def _image_kernel(q_ref, r_ref, aff_ref, entq_ref, entr_ref,
                  maskq_ref, maskr_ref, knnq_ref, knnr_ref):
    qn = _normalize(q_ref[0])
    rn = _normalize(r_ref[0])
    aff = jax.lax.dot_general(qn, rn, _DN,
                              preferred_element_type=jnp.float32)
    aff_ref[0] = aff

    ent_q = _entropy_row(aff.T)
    ent_r = _entropy_row(aff)
    entq_ref[0] = ent_q
    entr_ref[0] = ent_r

    ones_col = jnp.ones((N, 1), jnp.float32)
    ii = jax.lax.broadcasted_iota(jnp.int32, (N, N), 0)
    jj = jax.lax.broadcasted_iota(jnp.int32, (N, N), 1)
    maskq_ref[0] = _adaptive_mask_col(ent_q.T, ones_col, ii, jj).T
    maskr_ref[0] = _adaptive_mask_col(ent_r.T, ones_col, ii, jj).T

    knnq_ref[0] = _top3(aff, axis=1)
    knnr_ref[0] = _top3(aff, axis=0)


@jax.jit
def kernel(que_tokens, ref_tokens):
    grid = (B,)
    in_specs = [
        pl.BlockSpec((1, N, D), lambda b: (b, 0, 0)),
        pl.BlockSpec((1, N, D), lambda b: (b, 0, 0)),
    ]
    out_specs = [
        pl.BlockSpec((1, N, N), lambda b: (b, 0, 0)),
        pl.BlockSpec((1, 1, N), lambda b: (b, 0, 0)),
        pl.BlockSpec((1, 1, N), lambda b: (b, 0, 0)),
        pl.BlockSpec((1, 1, N), lambda b: (b, 0, 0)),
        pl.BlockSpec((1, 1, N), lambda b: (b, 0, 0)),
        pl.BlockSpec((1, N, K2), lambda b: (b, 0, 0)),
        pl.BlockSpec((1, N, K2), lambda b: (b, 0, 0)),
    ]
    out_shapes = [
        jax.ShapeDtypeStruct((B, N, N), jnp.float32),
        jax.ShapeDtypeStruct((B, 1, N), jnp.float32),
        jax.ShapeDtypeStruct((B, 1, N), jnp.float32),
        jax.ShapeDtypeStruct((B, 1, N), jnp.int32),
        jax.ShapeDtypeStruct((B, 1, N), jnp.int32),
        jax.ShapeDtypeStruct((B, N, K2), jnp.int32),
        jax.ShapeDtypeStruct((B, N, K2), jnp.int32),
    ]
    aff, ent_q, ent_r, mask_q, mask_r, knn_q2r, knn_r2q = pl.pallas_call(
        _image_kernel,
        grid=grid,
        in_specs=in_specs,
        out_specs=out_specs,
        out_shape=out_shapes,
        compiler_params=pltpu.CompilerParams(
            dimension_semantics=("parallel",),
        ),
    )(que_tokens, ref_tokens)
    return (aff,
            ent_q.reshape(B, N),
            ent_r.reshape(B, N),
            mask_q.reshape(B, N).astype(jnp.bool_),
            mask_r.reshape(B, N).astype(jnp.bool_),
            knn_q2r,
            knn_r2q)
